# Initial kernel scaffold; baseline (speedup 1.0000x reference)
import jax
import jax.numpy as jnp
from jax.experimental import pallas as pl


def kernel(z_category, categ_embed_weight):
    raise NotImplementedError("write your pallas kernel here")


# speedup vs baseline: 1.5477x; 1.5477x over previous
import functools

import jax
import jax.numpy as jnp
from jax import lax
from jax.experimental import pallas as pl
from jax.experimental.pallas import tpu as pltpu
from jax.experimental.pallas import tpu_sc as plsc

N_CLASS = 1000000
EMBED_DIM = 32
BATCH = 16384
FIELDS = 26
B_FLAT = BATCH * FIELDS

_NC = 2
_NS = 16
_NW = _NC * _NS

_B_PER_W = B_FLAT // _NW
_CHUNK = 1024
_NCHUNKS = _B_PER_W // _CHUNK


def _gather_body(idx_hbm, table_hbm, out_hbm, idx_v, rows_v, sem):
    c = lax.axis_index("c")
    s = lax.axis_index("s")
    wid = s * _NC + c
    base = wid * _B_PER_W

    def chunk(i, carry):
        off = base + i * _CHUNK
        pltpu.sync_copy(idx_hbm.at[pl.ds(off, _CHUNK)], idx_v)
        pltpu.async_copy(table_hbm.at[idx_v], rows_v, sem).wait()
        pltpu.sync_copy(rows_v, out_hbm.at[pl.ds(off, _CHUNK)])
        return carry

    lax.fori_loop(0, _NCHUNKS, chunk, 0)


@functools.partial(jax.jit, static_argnames=())
def _gather(idx_flat, table):
    mesh = plsc.VectorSubcoreMesh(core_axis_name="c", subcore_axis_name="s")
    kern = functools.partial(
        pl.kernel,
        mesh=mesh,
        out_type=jax.ShapeDtypeStruct((B_FLAT, EMBED_DIM), jnp.float32),
        scratch_types=[
            pltpu.VMEM((_CHUNK,), jnp.int32),
            pltpu.VMEM((_CHUNK, EMBED_DIM), jnp.float32),
            pltpu.SemaphoreType.DMA,
        ],
        compiler_params=pltpu.CompilerParams(use_tc_tiling_on_sc=False),
    )(_gather_body)
    return kern(idx_flat, table)


def kernel(z_category, categ_embed_weight):
    idx_flat = z_category.reshape(B_FLAT).astype(jnp.int32)
    rows = _gather(idx_flat, categ_embed_weight)
    return rows.reshape(BATCH, FIELDS, EMBED_DIM)


# speedup vs baseline: 1.5759x; 1.0182x over previous
import functools

import jax
import jax.numpy as jnp
from jax import lax
from jax.experimental import pallas as pl
from jax.experimental.pallas import tpu as pltpu
from jax.experimental.pallas import tpu_sc as plsc

N_CLASS = 1000000
EMBED_DIM = 32
BATCH = 16384
FIELDS = 26
B_FLAT = BATCH * FIELDS

_NC = 2
_NS = 16
_NW = _NC * _NS

_B_PER_W = B_FLAT // _NW
_CHUNK = 1664
_NCHUNKS = _B_PER_W // _CHUNK


def _gather_body(idx_hbm, table_hbm, out_hbm, idx_v, rows_v,
                 sem_g0, sem_g1, sem_w0, sem_w1):
    c = lax.axis_index("c")
    s = lax.axis_index("s")
    wid = s * _NC + c
    base = wid * _B_PER_W

    pltpu.sync_copy(idx_hbm.at[wid], idx_v)

    rows_b = (rows_v.at[0], rows_v.at[1])
    sem_g = (sem_g0, sem_g1)
    sem_w = (sem_w0, sem_w1)
    gathers = [None] * _NCHUNKS
    writes = [None] * _NCHUNKS

    for i in range(_NCHUNKS):
        b = i % 2
        if i >= 2:
            writes[i - 2].wait()
        gathers[i] = pltpu.async_copy(
            table_hbm.at[idx_v.at[i]], rows_b[b], sem_g[b])
        if i >= 1:
            pb = (i - 1) % 2
            gathers[i - 1].wait()
            writes[i - 1] = pltpu.async_copy(
                rows_b[pb], out_hbm.at[pl.ds(base + (i - 1) * _CHUNK, _CHUNK)],
                sem_w[pb])

    last = _NCHUNKS - 1
    gathers[last].wait()
    writes[last] = pltpu.async_copy(
        rows_b[last % 2],
        out_hbm.at[pl.ds(base + last * _CHUNK, _CHUNK)],
        sem_w[last % 2])
    writes[last - 1].wait()
    writes[last].wait()


@jax.jit
def _gather(idx3, table):
    mesh = plsc.VectorSubcoreMesh(core_axis_name="c", subcore_axis_name="s")
    kern = functools.partial(
        pl.kernel,
        mesh=mesh,
        out_type=jax.ShapeDtypeStruct((B_FLAT, EMBED_DIM), jnp.float32),
        scratch_types=[
            pltpu.VMEM((_NCHUNKS, _CHUNK), jnp.int32),
            pltpu.VMEM((2, _CHUNK, EMBED_DIM), jnp.float32),
            pltpu.SemaphoreType.DMA,
            pltpu.SemaphoreType.DMA,
            pltpu.SemaphoreType.DMA,
            pltpu.SemaphoreType.DMA,
        ],
        compiler_params=pltpu.CompilerParams(use_tc_tiling_on_sc=False),
    )(_gather_body)
    return kern(idx3, table)


def kernel(z_category, categ_embed_weight):
    idx3 = z_category.reshape(_NW, _NCHUNKS, _CHUNK).astype(jnp.int32)
    rows = _gather(idx3, categ_embed_weight)
    return rows.reshape(BATCH, FIELDS, EMBED_DIM)


# speedup vs baseline: 1.9681x; 1.2489x over previous
import functools

import jax
import jax.numpy as jnp
from jax import lax
from jax.experimental import pallas as pl
from jax.experimental.pallas import tpu as pltpu
from jax.experimental.pallas import tpu_sc as plsc

N_CLASS = 1000000
EMBED_DIM = 32
BATCH = 16384
FIELDS = 26

_NC = 2
_NS = 16
_NW = _NC * _NS

_B_PER_W = BATCH // _NW
_CB = 64
_NCHUNKS = _B_PER_W // _CB
_ROWS = _CB * FIELDS

_LANES = 128
_SUBL = 32
_IDXW = _ROWS + 16


def _gather_body(zpad_hbm, table_hbm, out_hbm, zstage0, zstage1,
                 idx_v0, idx_v1, rows_v0, rows_v1,
                 sem_g0, sem_g1, sem_w0, sem_w1):
    c = lax.axis_index("c")
    s = lax.axis_index("s")
    wid = s * _NC + c
    base = wid * _B_PER_W

    lane_iota = lax.iota(jnp.int32, 16)

    zstage = (zstage0, zstage1)
    idx_v = (idx_v0, idx_v1)
    rows_v = (rows_v0, rows_v1)
    sem_g = (sem_g0, sem_g1)
    sem_w = (sem_w0, sem_w1)
    gathers = [None] * _NCHUNKS
    writes = [None] * _NCHUNKS

    def start_chunk(i):
        p = i % 2
        b0 = base + i * _CB
        pltpu.sync_copy(
            zpad_hbm.at[pl.ds(b0, _CB), pl.ds(0, _SUBL)], zstage[p])

        def compact_row(r, carry):
            off = r * FIELDS
            idx_v[p][pl.ds(off, 16)] = zstage[p][r, pl.ds(0, 16)]
            idx_v[p][pl.ds(off + 10, 16)] = zstage[p][r, pl.ds(10, 16)]
            return carry

        lax.fori_loop(0, _CB, compact_row, 0)
        gathers[i] = pltpu.async_copy(
            table_hbm.at[idx_v[p].at[pl.ds(0, _ROWS)]],
            rows_v[p],
            sem_g[p])

    def drain_chunk(i):
        p = i % 2
        b0 = base + i * _CB
        gathers[i].wait()

        def write_row(r, carry):
            pltpu.async_copy(
                rows_v[p].at[pl.ds(r * FIELDS, FIELDS)],
                out_hbm.at[b0 + r, pl.ds(0, FIELDS), pl.ds(0, EMBED_DIM)],
                sem_w[p])
            return carry

        lax.fori_loop(0, _CB, write_row, 0)
        writes[i] = pltpu.make_async_copy(
            table_hbm.at[pl.ds(0, _ROWS)], rows_v[p], sem_w[p])

    for i in range(_NCHUNKS):
        if i >= 2:
            writes[i - 2].wait()
        start_chunk(i)
        if i >= 1:
            drain_chunk(i - 1)
    drain_chunk(_NCHUNKS - 1)
    writes[_NCHUNKS - 2].wait()
    writes[_NCHUNKS - 1].wait()


@jax.jit
def _gather(zpad, table):
    mesh = plsc.VectorSubcoreMesh(core_axis_name="c", subcore_axis_name="s")
    kern = functools.partial(
        pl.kernel,
        mesh=mesh,
        out_type=jax.ShapeDtypeStruct((BATCH, _SUBL, _LANES), jnp.float32),
        scratch_types=[
            pltpu.VMEM((_CB, _SUBL), jnp.int32),
            pltpu.VMEM((_CB, _SUBL), jnp.int32),
            pltpu.VMEM((_IDXW,), jnp.int32),
            pltpu.VMEM((_IDXW,), jnp.int32),
            pltpu.VMEM((_ROWS, EMBED_DIM), jnp.float32),
            pltpu.VMEM((_ROWS, EMBED_DIM), jnp.float32),
            pltpu.SemaphoreType.DMA,
            pltpu.SemaphoreType.DMA,
            pltpu.SemaphoreType.DMA,
            pltpu.SemaphoreType.DMA,
        ],
        compiler_params=pltpu.CompilerParams(use_tc_tiling_on_sc=False),
    )(_gather_body)
    return kern(zpad, table)


def kernel(z_category, categ_embed_weight):
    zpad = jnp.pad(z_category.astype(jnp.int32),
                   ((0, 0), (0, _LANES - FIELDS)))
    out_pad = _gather(zpad, categ_embed_weight)
    return out_pad[:, :FIELDS, :EMBED_DIM]


# speedup vs baseline: 1.9920x; 1.0122x over previous
import functools

import jax
import jax.numpy as jnp
from jax import lax
from jax.experimental import pallas as pl
from jax.experimental.pallas import tpu as pltpu
from jax.experimental.pallas import tpu_sc as plsc

N_CLASS = 1000000
EMBED_DIM = 32
BATCH = 16384
FIELDS = 26
B_FLAT = BATCH * FIELDS

_NC = 2
_NS = 16
_NW = _NC * _NS

_CHUNK = 1024
_NCHUNKS = B_FLAT // (_NW * _CHUNK)
_IDX_ROWS = B_FLAT // _CHUNK


def _gather_body(idx_hbm, table_hbm, out_hbm, idx_v, rows_v0, rows_v1,
                 sem_g0, sem_g1, sem_w0, sem_w1):
    c = lax.axis_index("c")
    s = lax.axis_index("s")
    wid = s * _NC + c
    r0 = wid * _NCHUNKS

    pltpu.sync_copy(idx_hbm.at[pl.ds(r0, _NCHUNKS)], idx_v)

    rows_v = (rows_v0, rows_v1)
    sem_g = (sem_g0, sem_g1)
    sem_w = (sem_w0, sem_w1)
    gathers = [None] * _NCHUNKS
    writes = [None] * _NCHUNKS

    def start_chunk(i):
        p = i % 2
        gathers[i] = pltpu.async_copy(
            table_hbm.at[idx_v.at[i]], rows_v[p], sem_g[p])

    def drain_chunk(i):
        p = i % 2
        pos0 = (r0 + i) * _CHUNK
        f = pos0 // BATCH
        b0 = pos0 - f * BATCH
        gathers[i].wait()
        writes[i] = pltpu.async_copy(
            rows_v[p],
            out_hbm.at[pl.ds(b0, _CHUNK), f, pl.ds(0, EMBED_DIM)],
            sem_w[p])

    for i in range(_NCHUNKS):
        if i >= 2:
            writes[i - 2].wait()
        start_chunk(i)
        if i >= 1:
            drain_chunk(i - 1)
    drain_chunk(_NCHUNKS - 1)
    writes[_NCHUNKS - 2].wait()
    writes[_NCHUNKS - 1].wait()


@jax.jit
def _gather(idx2d, table):
    mesh = plsc.VectorSubcoreMesh(core_axis_name="c", subcore_axis_name="s")
    kern = functools.partial(
        pl.kernel,
        mesh=mesh,
        out_type=jax.ShapeDtypeStruct((BATCH, 32, 128), jnp.float32),
        scratch_types=[
            pltpu.VMEM((_NCHUNKS, _CHUNK), jnp.int32),
            pltpu.VMEM((_CHUNK, EMBED_DIM), jnp.float32),
            pltpu.VMEM((_CHUNK, EMBED_DIM), jnp.float32),
            pltpu.SemaphoreType.DMA,
            pltpu.SemaphoreType.DMA,
            pltpu.SemaphoreType.DMA,
            pltpu.SemaphoreType.DMA,
        ],
        compiler_params=pltpu.CompilerParams(use_tc_tiling_on_sc=False),
    )(_gather_body)
    return kern(idx2d, table)


def kernel(z_category, categ_embed_weight):
    idx2d = z_category.astype(jnp.int32).T.reshape(_IDX_ROWS, _CHUNK)
    out_pad = _gather(idx2d, categ_embed_weight)
    return out_pad[:, :FIELDS, :EMBED_DIM]
